# baseline (speedup 1.0000x reference)
import functools

import jax
import jax.numpy as jnp
from jax import lax
from jax.experimental import pallas as pl
from jax.experimental.pallas import tpu as pltpu
from jax.experimental.pallas import tpu_sc as plsc

_BETA = 0.25
_N_TOK = 8192
_N_CODE = 8192
_D = 256
_BT = 1024
_BC = 1024
_T = _N_TOK // _BT
_C = _N_CODE // _BC
_CH1 = 2736
_CH2 = 5472
_CROSS1 = _CH1 // _BC
_CROSS2 = _CH2 // _BC
_OFF1 = _CH1 - _CROSS1 * _BC
_OFF2 = _CH2 - _CROSS2 * _BC


def _seg_minarg(dd, rows, mask):
    v = jnp.where(mask, dd, jnp.inf)
    m = jnp.min(v, axis=0)
    a = jnp.min(jnp.where(v == m[None, :], rows, _N_CODE), axis=0)
    return m, a


def _argmin_body(x_ref, e_ref, xsq_ref, esq_ref, idx_ref, loss_ref,
                 cmin_ref, carg_ref, facc_ref, fidx_ref, gmin_ref):
    t = pl.program_id(0)
    c = pl.program_id(1)

    x = x_ref[...]
    e = e_ref[...]
    s = lax.dot_general(
        e.astype(jnp.bfloat16), x.astype(jnp.bfloat16),
        (((1,), (1,)), ((), ())),
        preferred_element_type=jnp.float32,
    )
    dd = (xsq_ref[0, :][None, :] + esq_ref[:, 0:1]) - 2.0 * s
    rows = lax.broadcasted_iota(jnp.int32, (_BC, _BT), 0)

    m_all, a_all = _seg_minarg(dd, rows, jnp.full((_BC, _BT), True))
    a_all = a_all + c * _BC

    gprev = jnp.where(c == 0, jnp.inf, gmin_ref[...])
    gmin_ref[...] = jnp.minimum(gprev, m_all)

    is_cross = jnp.logical_or(c == _CROSS1, c == _CROSS2)

    @pl.when(jnp.logical_not(is_cross))
    def _():
        prev_m = jnp.where(c == 0, jnp.inf, cmin_ref[...])
        prev_a = jnp.where(c == 0, 0, carg_ref[...])
        better = m_all < prev_m
        cmin_ref[...] = jnp.where(better, m_all, prev_m)
        carg_ref[...] = jnp.where(better, a_all, prev_a)

        @pl.when(c == 0)
        def _():
            facc_ref[...] = jnp.full((_BT,), jnp.inf, jnp.float32)
            fidx_ref[...] = jnp.zeros((_BT,), jnp.int32)

    @pl.when(is_cross)
    def _():
        bnd = jnp.where(c == _CROSS1, _OFF1, _OFF2)
        mask_a = rows < bnd
        m_a, a_a = _seg_minarg(dd, rows, mask_a)
        m_b, a_b = _seg_minarg(dd, rows, jnp.logical_not(mask_a))
        pm = cmin_ref[...]
        pa = carg_ref[...]
        bet = m_a < pm
        chunk_m = jnp.where(bet, m_a, pm)
        chunk_a = jnp.where(bet, a_a + c * _BC, pa)
        fa = facc_ref[...]
        fb = chunk_m < fa
        facc_ref[...] = jnp.where(
            fb, chunk_m.astype(jnp.bfloat16).astype(jnp.float32), fa)
        fidx_ref[...] = jnp.where(fb, chunk_a, fidx_ref[...])
        cmin_ref[...] = m_b
        carg_ref[...] = a_b + c * _BC

    @pl.when(c == _C - 1)
    def _():
        cm = cmin_ref[...]
        fb = cm < facc_ref[...]
        idx_ref[0, 0, :] = jnp.where(fb, carg_ref[...], fidx_ref[...])
        part = jnp.sum(gmin_ref[...])
        prev_l = jnp.where(t == 0, jnp.zeros((1, 1), jnp.float32), loss_ref[...])
        loss_ref[...] = prev_l + part


def _argmin_call(flat_n, emb, xsq8, esq8):
    return pl.pallas_call(
        _argmin_body,
        grid=(_T, _C),
        in_specs=[
            pl.BlockSpec((_BT, _D), lambda t, c: (t, 0)),
            pl.BlockSpec((_BC, _D), lambda t, c: (c, 0)),
            pl.BlockSpec((8, _BT), lambda t, c: (0, t)),
            pl.BlockSpec((_BC, 8), lambda t, c: (c, 0)),
        ],
        out_specs=[
            pl.BlockSpec((1, 1, _BT), lambda t, c: (t, 0, 0)),
            pl.BlockSpec((1, 1), lambda t, c: (0, 0)),
        ],
        out_shape=[
            jax.ShapeDtypeStruct((_T, 1, _BT), jnp.int32),
            jax.ShapeDtypeStruct((1, 1), jnp.float32),
        ],
        scratch_shapes=[
            pltpu.VMEM((_BT,), jnp.float32),
            pltpu.VMEM((_BT,), jnp.int32),
            pltpu.VMEM((_BT,), jnp.float32),
            pltpu.VMEM((_BT,), jnp.int32),
            pltpu.VMEM((_BT,), jnp.float32),
        ],
    )(flat_n, emb, xsq8, esq8)


@functools.cache
def _make_sc_gather():
    info = plsc.get_sparse_core_info()
    nc, ns = info.num_cores, info.num_subcores
    nw = nc * ns
    bpw = _N_TOK // nw
    mesh = plsc.VectorSubcoreMesh(core_axis_name="c", subcore_axis_name="s")

    @functools.partial(
        pl.kernel,
        mesh=mesh,
        out_type=jax.ShapeDtypeStruct((_N_TOK, _D), jnp.float32),
        scratch_types=[
            pltpu.VMEM((bpw,), jnp.int32),
            pltpu.VMEM((bpw, _D), jnp.float32),
            pltpu.SemaphoreType.DMA,
        ],
    )
    def gather_k(table_hbm, idx_hbm, out_hbm, idx_v, rows_v, sem):
        wid = lax.axis_index("s") * nc + lax.axis_index("c")
        base = wid * bpw
        pltpu.sync_copy(idx_hbm.at[pl.ds(base, bpw)], idx_v)
        pltpu.async_copy(table_hbm.at[idx_v], rows_v, sem).wait()
        pltpu.sync_copy(rows_v, out_hbm.at[pl.ds(base, bpw)])

    return gather_k


def kernel(hidden_state, codebook):
    hs_t = jnp.transpose(hidden_state, (0, 2, 3, 1))
    n4 = jnp.clip(jnp.linalg.norm(hs_t, axis=-1, keepdims=True), 1e-12)
    hs_n = hs_t / n4
    flat_n = hs_n.reshape(_N_TOK, _D)
    emb = codebook / jnp.clip(jnp.linalg.norm(codebook, axis=-1, keepdims=True), 1e-12)
    xsq = jnp.sum(hs_n ** 2, axis=3).reshape(-1)
    esq = jnp.sum(emb ** 2, axis=1)
    xsq8 = jnp.broadcast_to(xsq[None, :], (8, _N_TOK))
    esq8 = jnp.broadcast_to(esq[:, None], (_N_CODE, 8))

    idx3, loss_raw = _argmin_call(flat_n, emb, xsq8, esq8)
    idx = idx3.reshape(_N_TOK)
    quant_flat = _make_sc_gather()(emb, idx)
    hq = jnp.transpose(quant_flat.reshape(8, 32, 32, _D), (0, 3, 1, 2))
    loss = loss_raw[0, 0] * ((1.0 + _BETA) / (_N_TOK * _D))
    return hq, loss, idx


# speedup vs baseline: 1.1474x; 1.1474x over previous
import functools

import jax
import jax.numpy as jnp
from jax import lax
from jax.experimental import pallas as pl
from jax.experimental.pallas import tpu as pltpu
from jax.experimental.pallas import tpu_sc as plsc

_BETA = 0.25
_N_TOK = 8192
_N_CODE = 8192
_D = 256
_BT = 2048
_BC = 1024
_T = _N_TOK // _BT
_C = _N_CODE // _BC
_CH1 = 2736
_CH2 = 5472
_CROSS1 = _CH1 // _BC
_CROSS2 = _CH2 // _BC
_OFF1 = _CH1 - _CROSS1 * _BC
_OFF2 = _CH2 - _CROSS2 * _BC


def _seg_minarg(dd, rows, mask):
    v = jnp.where(mask, dd, jnp.inf)
    m = jnp.min(v, axis=0)
    a = jnp.min(jnp.where(v == m[None, :], rows, _N_CODE), axis=0)
    return m, a


def _argmin_body(x_ref, e_ref, xsq_ref, esq_ref, idx_ref, loss_ref,
                 cmin_ref, carg_ref, facc_ref, fidx_ref, gmin_ref):
    t = pl.program_id(0)
    c = pl.program_id(1)

    x = x_ref[...]
    e = e_ref[...]
    s = lax.dot_general(
        e.astype(jnp.bfloat16), x.astype(jnp.bfloat16),
        (((1,), (1,)), ((), ())),
        preferred_element_type=jnp.float32,
    )
    dd = (xsq_ref[0, :][None, :] + esq_ref[:, 0:1]) - 2.0 * s
    rows = lax.broadcasted_iota(jnp.int32, (_BC, _BT), 0)

    is_cross = jnp.logical_or(c == _CROSS1, c == _CROSS2)

    @pl.when(jnp.logical_not(is_cross))
    def _():
        m_all, a_all = _seg_minarg(dd, rows, jnp.full((_BC, _BT), True))
        a_all = a_all + c * _BC
        gprev = jnp.where(c == 0, jnp.inf, gmin_ref[...])
        gmin_ref[...] = jnp.minimum(gprev, m_all)
        prev_m = jnp.where(c == 0, jnp.inf, cmin_ref[...])
        prev_a = jnp.where(c == 0, 0, carg_ref[...])
        better = m_all < prev_m
        cmin_ref[...] = jnp.where(better, m_all, prev_m)
        carg_ref[...] = jnp.where(better, a_all, prev_a)

        @pl.when(c == 0)
        def _():
            facc_ref[...] = jnp.full((_BT,), jnp.inf, jnp.float32)
            fidx_ref[...] = jnp.zeros((_BT,), jnp.int32)

    @pl.when(is_cross)
    def _():
        bnd = jnp.where(c == _CROSS1, _OFF1, _OFF2)
        mask_a = rows < bnd
        m_a, a_a = _seg_minarg(dd, rows, mask_a)
        m_b, a_b = _seg_minarg(dd, rows, jnp.logical_not(mask_a))
        gmin_ref[...] = jnp.minimum(gmin_ref[...], jnp.minimum(m_a, m_b))
        pm = cmin_ref[...]
        pa = carg_ref[...]
        bet = m_a < pm
        chunk_m = jnp.where(bet, m_a, pm)
        chunk_a = jnp.where(bet, a_a + c * _BC, pa)
        fa = facc_ref[...]
        fb = chunk_m < fa
        facc_ref[...] = jnp.where(
            fb, chunk_m.astype(jnp.bfloat16).astype(jnp.float32), fa)
        fidx_ref[...] = jnp.where(fb, chunk_a, fidx_ref[...])
        cmin_ref[...] = m_b
        carg_ref[...] = a_b + c * _BC

    @pl.when(c == _C - 1)
    def _():
        cm = cmin_ref[...]
        fb = cm < facc_ref[...]
        idx_ref[0, 0, :] = jnp.where(fb, carg_ref[...], fidx_ref[...])
        part = jnp.sum(gmin_ref[...])
        prev_l = jnp.where(t == 0, jnp.zeros((1, 1), jnp.float32), loss_ref[...])
        loss_ref[...] = prev_l + part


def _argmin_call(flat_n, emb, xsq8, esq8):
    return pl.pallas_call(
        _argmin_body,
        grid=(_T, _C),
        in_specs=[
            pl.BlockSpec((_BT, _D), lambda t, c: (t, 0)),
            pl.BlockSpec((_BC, _D), lambda t, c: (c, 0)),
            pl.BlockSpec((8, _BT), lambda t, c: (0, t)),
            pl.BlockSpec((_BC, 8), lambda t, c: (c, 0)),
        ],
        out_specs=[
            pl.BlockSpec((1, 1, _BT), lambda t, c: (t, 0, 0)),
            pl.BlockSpec((1, 1), lambda t, c: (0, 0)),
        ],
        out_shape=[
            jax.ShapeDtypeStruct((_T, 1, _BT), jnp.int32),
            jax.ShapeDtypeStruct((1, 1), jnp.float32),
        ],
        scratch_shapes=[
            pltpu.VMEM((_BT,), jnp.float32),
            pltpu.VMEM((_BT,), jnp.int32),
            pltpu.VMEM((_BT,), jnp.float32),
            pltpu.VMEM((_BT,), jnp.int32),
            pltpu.VMEM((_BT,), jnp.float32),
        ],
    )(flat_n, emb, xsq8, esq8)


@functools.cache
def _make_sc_gather():
    info = plsc.get_sparse_core_info()
    nc, ns = info.num_cores, info.num_subcores
    nw = nc * ns
    bpw = _N_TOK // nw
    mesh = plsc.VectorSubcoreMesh(core_axis_name="c", subcore_axis_name="s")

    @functools.partial(
        pl.kernel,
        mesh=mesh,
        out_type=jax.ShapeDtypeStruct((_N_TOK, _D), jnp.float32),
        scratch_types=[
            pltpu.VMEM((bpw,), jnp.int32),
            pltpu.VMEM((bpw, _D), jnp.float32),
            pltpu.SemaphoreType.DMA,
        ],
    )
    def gather_k(table_hbm, idx_hbm, out_hbm, idx_v, rows_v, sem):
        wid = lax.axis_index("s") * nc + lax.axis_index("c")
        base = wid * bpw
        pltpu.sync_copy(idx_hbm.at[pl.ds(base, bpw)], idx_v)
        pltpu.async_copy(table_hbm.at[idx_v], rows_v, sem).wait()
        pltpu.sync_copy(rows_v, out_hbm.at[pl.ds(base, bpw)])

    return gather_k


def kernel(hidden_state, codebook):
    hs_t = jnp.transpose(hidden_state, (0, 2, 3, 1))
    n4 = jnp.clip(jnp.linalg.norm(hs_t, axis=-1, keepdims=True), 1e-12)
    hs_n = hs_t / n4
    flat_n = hs_n.reshape(_N_TOK, _D)
    emb = codebook / jnp.clip(jnp.linalg.norm(codebook, axis=-1, keepdims=True), 1e-12)
    xsq = jnp.sum(hs_n ** 2, axis=3).reshape(-1)
    esq = jnp.sum(emb ** 2, axis=1)
    xsq8 = jnp.broadcast_to(xsq[None, :], (8, _N_TOK))
    esq8 = jnp.broadcast_to(esq[:, None], (_N_CODE, 8))

    idx3, loss_raw = _argmin_call(flat_n, emb, xsq8, esq8)
    idx = idx3.reshape(_N_TOK)
    quant_flat = _make_sc_gather()(emb, idx)
    hq = jnp.transpose(quant_flat.reshape(8, 32, 32, _D), (0, 3, 1, 2))
    loss = loss_raw[0, 0] * ((1.0 + _BETA) / (_N_TOK * _D))
    return hq, loss, idx
